# baseline (speedup 1.0000x reference)
import functools

import jax
import jax.numpy as jnp
from jax import lax
from jax.experimental import pallas as pl
from jax.experimental.pallas import tpu as pltpu
from jax.experimental.pallas import tpu_sc as plsc

N = 10000
E = 320000
D_IN = 128
D_H = 128
D_OUT = 40
D3 = 64

NCORE = 2
NSUB = 16
CHUNK = 256
NBUF = 4
SEG = 20
FAST_CORE = 1
FSEG = 2
CPW_FAST = FSEG * SEG
CPW_SLOW = FSEG * SEG
TOTALC = NSUB * (CPW_FAST + CPW_SLOW)
EPAD = TOTALC * CHUNK
NPAD = 10112
RPS = NPAD // NSUB

_MESH = plsc.VectorSubcoreMesh(
    core_axis_name="c", subcore_axis_name="s",
    num_cores=NCORE, num_subcores=NSUB)


def _make_agg(D):
    @functools.partial(
        pl.kernel,
        out_type=jax.ShapeDtypeStruct((2 * NPAD, D), jnp.float32),
        mesh=_MESH,
        compiler_params=pltpu.CompilerParams(use_tc_tiling_on_sc=False),
        scratch_types=[
            pltpu.VMEM((SEG, CHUNK), jnp.int32),
            pltpu.VMEM((SEG, CHUNK), jnp.int32),
            pltpu.VMEM((CHUNK, D), jnp.float32),
            pltpu.VMEM_SHARED((NPAD, D), jnp.float32),
            pltpu.SemaphoreType.DMA,
        ],
    )
    def agg(h_hbm, src_hbm, dst_hbm, zeros_hbm, out_hbm,
            src_v, dst_v, rows_v, acc, gsem):
        c = lax.axis_index("c")
        s = lax.axis_index("s")
        base = (c * NSUB + s) * CPW_FAST
        r0 = s * RPS
        pltpu.sync_copy(zeros_hbm.at[pl.ds(r0, RPS)], acc.at[pl.ds(r0, RPS)])

        def run_segment(segbase):
            pltpu.sync_copy(src_hbm.at[pl.ds(segbase, SEG)], src_v)
            pltpu.sync_copy(dst_hbm.at[pl.ds(segbase, SEG)], dst_v)

            def body(j, carry):
                pltpu.async_copy(h_hbm.at[src_v.at[j]], rows_v, gsem).wait()
                pltpu.sync_copy(rows_v, acc.at[dst_v.at[j]], add=True)
                return carry

            lax.fori_loop(0, SEG, body, 0)

        plsc.subcore_barrier()
        for seg_i in range(FSEG):
            run_segment(base + seg_i * SEG)
        plsc.subcore_barrier()
        pltpu.sync_copy(acc.at[pl.ds(r0, RPS)],
                        out_hbm.at[pl.ds(c * NPAD + r0, RPS)])

    return agg


_agg128 = _make_agg(D_H)
_agg64 = _make_agg(D3)


@functools.partial(
    pl.kernel,
    out_type=(jax.ShapeDtypeStruct((2 * NPAD, 16), jnp.float32),
              jax.ShapeDtypeStruct((2 * NPAD, 16), jnp.float32)),
    mesh=_MESH,
    compiler_params=pltpu.CompilerParams(use_tc_tiling_on_sc=False),
    scratch_types=[
        pltpu.VMEM((SEG, CHUNK), jnp.int32),
        pltpu.VMEM((SEG, CHUNK), jnp.int32),
        pltpu.VMEM((CHUNK, 16), jnp.float32),
        pltpu.VMEM_SHARED((NPAD, 16), jnp.float32),
        pltpu.VMEM_SHARED((NPAD, 16), jnp.float32),
        [pltpu.SemaphoreType.DMA for _ in range(NBUF)],
        [pltpu.SemaphoreType.DMA for _ in range(NBUF)],
    ],
)
def _deg_kernel(src_hbm, dst_hbm, zeros_hbm, ones_hbm,
                outdeg_hbm, indeg_hbm,
                src_v, dst_v, ones_v, acc_a, acc_b, asem, bsem):
    c = lax.axis_index("c")
    s = lax.axis_index("s")
    base = (c * NSUB + s) * CPW_FAST
    r0 = s * RPS
    pltpu.sync_copy(zeros_hbm.at[pl.ds(r0, RPS)], acc_a.at[pl.ds(r0, RPS)])
    pltpu.sync_copy(zeros_hbm.at[pl.ds(r0, RPS)], acc_b.at[pl.ds(r0, RPS)])
    pltpu.sync_copy(ones_hbm, ones_v)
    plsc.subcore_barrier()

    def run_segment(segbase):
        pltpu.sync_copy(src_hbm.at[pl.ds(segbase, SEG)], src_v)
        pltpu.sync_copy(dst_hbm.at[pl.ds(segbase, SEG)], dst_v)

        def body(g, carry):
            for b in range(NBUF):
                @pl.when(g > 0)
                def _():
                    pltpu.make_async_copy(
                        ones_v, acc_a.at[src_v.at[0]], asem[b]).wait()
                    pltpu.make_async_copy(
                        ones_v, acc_b.at[dst_v.at[0]], bsem[b]).wait()
                j = g * NBUF + b
                pltpu.async_copy(ones_v, acc_a.at[src_v.at[j]], asem[b],
                                 add=True)
                pltpu.async_copy(ones_v, acc_b.at[dst_v.at[j]], bsem[b],
                                 add=True)
            return carry

        lax.fori_loop(0, SEG // NBUF, body, 0)
        for b in range(NBUF):
            pltpu.make_async_copy(
                ones_v, acc_a.at[src_v.at[0]], asem[b]).wait()
            pltpu.make_async_copy(
                ones_v, acc_b.at[dst_v.at[0]], bsem[b]).wait()

    for seg_i in range(FSEG):
        run_segment(base + seg_i * SEG)
    plsc.subcore_barrier()
    pltpu.sync_copy(acc_a.at[pl.ds(r0, RPS)],
                    outdeg_hbm.at[pl.ds(c * NPAD + r0, RPS)])
    pltpu.sync_copy(acc_b.at[pl.ds(r0, RPS)],
                    indeg_hbm.at[pl.ds(c * NPAD + r0, RPS)])


_PREC = jax.lax.Precision.HIGHEST


def _tc_norms_body(degs_ref, degd_ref, ns_ref, nd_ref):
    out_deg = degs_ref[:NPAD, 0:1] + degs_ref[NPAD:, 0:1]
    in_deg = degd_ref[:NPAD, 0:1] + degd_ref[NPAD:, 0:1]
    ns_ref[...] = lax.rsqrt(jnp.maximum(out_deg, 1.0))
    nd_ref[...] = lax.rsqrt(jnp.maximum(in_deg, 1.0))


def _tc_norms(degs, degd):
    return pl.pallas_call(
        _tc_norms_body,
        out_shape=(jax.ShapeDtypeStruct((NPAD, 1), jnp.float32),
                   jax.ShapeDtypeStruct((NPAD, 1), jnp.float32)),
    )(degs, degd)


def _tc_z1_body(x_ref, w1_ref, ns_ref, z_ref):
    z = jnp.dot(x_ref[...], w1_ref[...],
                preferred_element_type=jnp.float32, precision=_PREC)
    z_ref[...] = z * ns_ref[...]


def _tc_z1(x, w1, ns):
    return pl.pallas_call(
        _tc_z1_body,
        out_shape=jax.ShapeDtypeStruct((NPAD, D_H), jnp.float32),
    )(x, w1, ns)


def _tc_mid_body(agg_ref, nd_ref, b_ref, w_ref, ns_ref, z_ref):
    agg = agg_ref[:NPAD, :] + agg_ref[NPAD:, :]
    h = jnp.maximum(agg * nd_ref[...] + b_ref[...][None, :], 0.0)
    mask = lax.broadcasted_iota(jnp.int32, (NPAD, 1), 0) < N
    h = jnp.where(mask, h, 0.0)
    mu = jnp.sum(h) / (N * D_H)
    d = h - mu
    var = jnp.sum(jnp.where(mask, d * d, 0.0)) / (N * D_H)
    hn = jnp.where(mask, d * lax.rsqrt(var + 1e-5), 0.0)
    z = jnp.dot(hn, w_ref[...],
                preferred_element_type=jnp.float32, precision=_PREC)
    z_ref[...] = z * ns_ref[...]


def _tc_mid(agg, nd, b, w, ns):
    return pl.pallas_call(
        _tc_mid_body,
        out_shape=jax.ShapeDtypeStruct((NPAD, w.shape[1]), jnp.float32),
    )(agg, nd, b, w, ns)


def _tc_final_body(agg_ref, nd_ref, b_ref, out_ref):
    agg = agg_ref[:NPAD, :] + agg_ref[NPAD:, :]
    out_ref[...] = agg * nd_ref[...] + b_ref[...][None, :]


def _tc_final(agg, nd, b):
    return pl.pallas_call(
        _tc_final_body,
        out_shape=jax.ShapeDtypeStruct((NPAD, D3), jnp.float32),
    )(agg, nd, b)


def kernel(features, edge_index, W1, b1, W2, b2, W3, b3):
    src = edge_index[0]
    dst = edge_index[1]
    pad = jnp.full((EPAD - E,), N, dtype=jnp.int32)
    src3 = jnp.concatenate([src, pad]).reshape(TOTALC, CHUNK)
    dst3 = jnp.concatenate([dst, pad]).reshape(TOTALC, CHUNK)

    x = jnp.zeros((NPAD, D_IN), jnp.float32).at[:N].set(features)
    zeros128 = jnp.zeros((NPAD, D_H), jnp.float32)
    zeros64 = jnp.zeros((NPAD, D3), jnp.float32)
    zeros16 = jnp.zeros((NPAD, 16), jnp.float32)
    ones16 = jnp.ones((CHUNK, 16), jnp.float32)
    W3p = jnp.zeros((D_H, D3), jnp.float32).at[:, :D_OUT].set(W3)
    b3p = jnp.zeros((D3,), jnp.float32).at[:D_OUT].set(b3)

    degs, degd = _deg_kernel(src3, dst3, zeros16, ones16)
    ns, nd = _tc_norms(degs, degd)
    z1 = _tc_z1(x, W1, ns)
    a1 = _agg128(z1, src3, dst3, zeros128)
    z2 = _tc_mid(a1, nd, b1, W2, ns)
    a2 = _agg128(z2, src3, dst3, zeros128)
    z3 = _tc_mid(a2, nd, b2, W3p, ns)
    a3 = _agg64(z3, src3, dst3, zeros64)
    outp = _tc_final(a3, nd, b3p)
    return outp[:N, :D_OUT]


# speedup vs baseline: 2.0490x; 2.0490x over previous
import functools

import jax
import jax.numpy as jnp
from jax import lax
from jax.experimental import pallas as pl
from jax.experimental.pallas import tpu as pltpu
from jax.experimental.pallas import tpu_sc as plsc

N = 10000
E = 320000
D_IN = 128
D_H = 128
D_OUT = 40
D3 = 64

NCORE = 2
NSUB = 16
CHUNK = 256
NBUF = 4
SEG = 20
FAST_CORE = 1
FSEG = 2
CPW_FAST = FSEG * SEG
CPW_SLOW = FSEG * SEG
TOTALC = NSUB * (CPW_FAST + CPW_SLOW)
EPAD = TOTALC * CHUNK
NPAD = 10112
RPS = NPAD // NSUB

_MESH = plsc.VectorSubcoreMesh(
    core_axis_name="c", subcore_axis_name="s",
    num_cores=NCORE, num_subcores=NSUB)


def _make_agg(D):
    npass = D // D3

    @functools.partial(
        pl.kernel,
        out_type=jax.ShapeDtypeStruct((2 * NPAD, D), jnp.float32),
        mesh=_MESH,
        compiler_params=pltpu.CompilerParams(use_tc_tiling_on_sc=False),
        scratch_types=[
            pltpu.VMEM((CPW_FAST, CHUNK), jnp.int32),
            pltpu.VMEM((CPW_FAST, CHUNK), jnp.int32),
            pltpu.VMEM((CHUNK, D3), jnp.float32),
            pltpu.VMEM_SHARED((NPAD, D3), jnp.float32),
            pltpu.VMEM_SHARED((NPAD, D3), jnp.float32),
            pltpu.SemaphoreType.DMA,
        ],
    )
    def agg(z_hbm, src_hbm, dst_hbm, out_hbm,
            src_v, dst_v, rows_v, hs, acc, gsem):
        c = lax.axis_index("c")
        s = lax.axis_index("s")
        base = (c * NSUB + s) * CPW_FAST
        r0 = s * RPS
        pltpu.sync_copy(src_hbm.at[pl.ds(base, CPW_FAST)], src_v)
        pltpu.sync_copy(dst_hbm.at[pl.ds(base, CPW_FAST)], dst_v)

        for p in range(npass):
            def zrow(r, carry):
                for k in range(D3 // 16):
                    rows_v[r, pl.ds(k * 16, 16)] = jnp.zeros((16,),
                                                             jnp.float32)
                return carry

            lax.fori_loop(0, CHUNK, zrow, 0)
            if npass == 1:
                pltpu.sync_copy(z_hbm.at[pl.ds(r0, RPS)],
                                hs.at[pl.ds(r0, RPS)])
            else:
                pltpu.sync_copy(
                    z_hbm.at[pl.ds(r0, RPS), pl.ds(p * D3, D3)],
                    hs.at[pl.ds(r0, RPS)])
            full = RPS // CHUNK
            for q in range(full):
                pltpu.sync_copy(rows_v, acc.at[pl.ds(r0 + q * CHUNK, CHUNK)])
            rem = RPS - full * CHUNK
            if rem:
                pltpu.sync_copy(rows_v.at[pl.ds(0, rem)],
                                acc.at[pl.ds(r0 + full * CHUNK, rem)])
            plsc.subcore_barrier()

            def body(j, carry):
                pltpu.async_copy(hs.at[src_v.at[j]], rows_v, gsem).wait()
                pltpu.sync_copy(rows_v, acc.at[dst_v.at[j]], add=True)
                return carry

            lax.fori_loop(0, CPW_FAST, body, 0)
            plsc.subcore_barrier()
            if npass == 1:
                pltpu.sync_copy(acc.at[pl.ds(r0, RPS)],
                                out_hbm.at[pl.ds(c * NPAD + r0, RPS)])
            else:
                pltpu.sync_copy(
                    acc.at[pl.ds(r0, RPS)],
                    out_hbm.at[pl.ds(c * NPAD + r0, RPS), pl.ds(p * D3, D3)])

    return agg


_agg128 = _make_agg(D_H)
_agg64 = _make_agg(D3)


@functools.partial(
    pl.kernel,
    out_type=(jax.ShapeDtypeStruct((2 * NPAD, 16), jnp.float32),
              jax.ShapeDtypeStruct((2 * NPAD, 16), jnp.float32)),
    mesh=_MESH,
    compiler_params=pltpu.CompilerParams(use_tc_tiling_on_sc=False),
    scratch_types=[
        pltpu.VMEM((SEG, CHUNK), jnp.int32),
        pltpu.VMEM((SEG, CHUNK), jnp.int32),
        pltpu.VMEM((CHUNK, 16), jnp.float32),
        pltpu.VMEM_SHARED((NPAD, 16), jnp.float32),
        pltpu.VMEM_SHARED((NPAD, 16), jnp.float32),
        [pltpu.SemaphoreType.DMA for _ in range(NBUF)],
        [pltpu.SemaphoreType.DMA for _ in range(NBUF)],
    ],
)
def _deg_kernel(src_hbm, dst_hbm, zeros_hbm, ones_hbm,
                outdeg_hbm, indeg_hbm,
                src_v, dst_v, ones_v, acc_a, acc_b, asem, bsem):
    c = lax.axis_index("c")
    s = lax.axis_index("s")
    base = (c * NSUB + s) * CPW_FAST
    r0 = s * RPS
    pltpu.sync_copy(zeros_hbm.at[pl.ds(r0, RPS)], acc_a.at[pl.ds(r0, RPS)])
    pltpu.sync_copy(zeros_hbm.at[pl.ds(r0, RPS)], acc_b.at[pl.ds(r0, RPS)])
    pltpu.sync_copy(ones_hbm, ones_v)
    plsc.subcore_barrier()

    def run_segment(segbase):
        pltpu.sync_copy(src_hbm.at[pl.ds(segbase, SEG)], src_v)
        pltpu.sync_copy(dst_hbm.at[pl.ds(segbase, SEG)], dst_v)

        def body(g, carry):
            for b in range(NBUF):
                @pl.when(g > 0)
                def _():
                    pltpu.make_async_copy(
                        ones_v, acc_a.at[src_v.at[0]], asem[b]).wait()
                    pltpu.make_async_copy(
                        ones_v, acc_b.at[dst_v.at[0]], bsem[b]).wait()
                j = g * NBUF + b
                pltpu.async_copy(ones_v, acc_a.at[src_v.at[j]], asem[b],
                                 add=True)
                pltpu.async_copy(ones_v, acc_b.at[dst_v.at[j]], bsem[b],
                                 add=True)
            return carry

        lax.fori_loop(0, SEG // NBUF, body, 0)
        for b in range(NBUF):
            pltpu.make_async_copy(
                ones_v, acc_a.at[src_v.at[0]], asem[b]).wait()
            pltpu.make_async_copy(
                ones_v, acc_b.at[dst_v.at[0]], bsem[b]).wait()

    for seg_i in range(FSEG):
        run_segment(base + seg_i * SEG)
    plsc.subcore_barrier()
    pltpu.sync_copy(acc_a.at[pl.ds(r0, RPS)],
                    outdeg_hbm.at[pl.ds(c * NPAD + r0, RPS)])
    pltpu.sync_copy(acc_b.at[pl.ds(r0, RPS)],
                    indeg_hbm.at[pl.ds(c * NPAD + r0, RPS)])


_PREC = jax.lax.Precision.HIGHEST


def _tc_norms_body(degs_ref, degd_ref, ns_ref, nd_ref):
    out_deg = degs_ref[:NPAD, 0:1] + degs_ref[NPAD:, 0:1]
    in_deg = degd_ref[:NPAD, 0:1] + degd_ref[NPAD:, 0:1]
    ns_ref[...] = lax.rsqrt(jnp.maximum(out_deg, 1.0))
    nd_ref[...] = lax.rsqrt(jnp.maximum(in_deg, 1.0))


def _tc_norms(degs, degd):
    return pl.pallas_call(
        _tc_norms_body,
        out_shape=(jax.ShapeDtypeStruct((NPAD, 1), jnp.float32),
                   jax.ShapeDtypeStruct((NPAD, 1), jnp.float32)),
    )(degs, degd)


def _tc_z1_body(x_ref, w1_ref, ns_ref, z_ref):
    z = jnp.dot(x_ref[...], w1_ref[...],
                preferred_element_type=jnp.float32, precision=_PREC)
    z_ref[...] = z * ns_ref[...]


def _tc_z1(x, w1, ns):
    return pl.pallas_call(
        _tc_z1_body,
        out_shape=jax.ShapeDtypeStruct((NPAD, D_H), jnp.float32),
    )(x, w1, ns)


def _tc_mid_body(agg_ref, nd_ref, b_ref, w_ref, ns_ref, z_ref):
    agg = agg_ref[:NPAD, :] + agg_ref[NPAD:, :]
    h = jnp.maximum(agg * nd_ref[...] + b_ref[...][None, :], 0.0)
    mask = lax.broadcasted_iota(jnp.int32, (NPAD, 1), 0) < N
    h = jnp.where(mask, h, 0.0)
    mu = jnp.sum(h) / (N * D_H)
    d = h - mu
    var = jnp.sum(jnp.where(mask, d * d, 0.0)) / (N * D_H)
    hn = jnp.where(mask, d * lax.rsqrt(var + 1e-5), 0.0)
    z = jnp.dot(hn, w_ref[...],
                preferred_element_type=jnp.float32, precision=_PREC)
    z_ref[...] = z * ns_ref[...]


def _tc_mid(agg, nd, b, w, ns):
    return pl.pallas_call(
        _tc_mid_body,
        out_shape=jax.ShapeDtypeStruct((NPAD, w.shape[1]), jnp.float32),
    )(agg, nd, b, w, ns)


def _tc_final_body(agg_ref, nd_ref, b_ref, out_ref):
    agg = agg_ref[:NPAD, :] + agg_ref[NPAD:, :]
    out_ref[...] = agg * nd_ref[...] + b_ref[...][None, :]


def _tc_final(agg, nd, b):
    return pl.pallas_call(
        _tc_final_body,
        out_shape=jax.ShapeDtypeStruct((NPAD, D3), jnp.float32),
    )(agg, nd, b)


def kernel(features, edge_index, W1, b1, W2, b2, W3, b3):
    src = edge_index[0]
    dst = edge_index[1]
    pad = jnp.full((EPAD - E,), N, dtype=jnp.int32)
    src3 = jnp.concatenate([src, pad]).reshape(TOTALC, CHUNK)
    dst3 = jnp.concatenate([dst, pad]).reshape(TOTALC, CHUNK)

    x = jnp.zeros((NPAD, D_IN), jnp.float32).at[:N].set(features)
    zeros16 = jnp.zeros((NPAD, 16), jnp.float32)
    ones16 = jnp.ones((CHUNK, 16), jnp.float32)
    W3p = jnp.zeros((D_H, D3), jnp.float32).at[:, :D_OUT].set(W3)
    b3p = jnp.zeros((D3,), jnp.float32).at[:D_OUT].set(b3)

    degs, degd = _deg_kernel(src3, dst3, zeros16, ones16)
    ns, nd = _tc_norms(degs, degd)
    z1 = _tc_z1(x, W1, ns)
    a1 = _agg128(z1, src3, dst3)
    z2 = _tc_mid(a1, nd, b1, W2, ns)
    a2 = _agg128(z2, src3, dst3)
    z3 = _tc_mid(a2, nd, b2, W3p, ns)
    a3 = _agg64(z3, src3, dst3)
    outp = _tc_final(a3, nd, b3p)
    return outp[:N, :D_OUT]


# speedup vs baseline: 2.5844x; 1.2613x over previous
import functools

import jax
import jax.numpy as jnp
from jax import lax
from jax.experimental import pallas as pl
from jax.experimental.pallas import tpu as pltpu
from jax.experimental.pallas import tpu_sc as plsc

N = 10000
E = 320000
D_IN = 128
D_H = 128
D_OUT = 40
D3 = 64

NCORE = 2
NSUB = 16
CHUNK = 256
NBUF = 4
SEG = 20
FAST_CORE = 1
FSEG = 2
CPW_FAST = FSEG * SEG
CPW_SLOW = FSEG * SEG
TOTALC = NSUB * (CPW_FAST + CPW_SLOW)
EPAD = TOTALC * CHUNK
NPAD = 10112
RPS = NPAD // NSUB

_MESH = plsc.VectorSubcoreMesh(
    core_axis_name="c", subcore_axis_name="s",
    num_cores=NCORE, num_subcores=NSUB)


def _make_agg(D):
    npass = D // D3

    @functools.partial(
        pl.kernel,
        out_type=jax.ShapeDtypeStruct((2 * NPAD, D), jnp.float32),
        mesh=_MESH,
        compiler_params=pltpu.CompilerParams(use_tc_tiling_on_sc=False),
        scratch_types=[
            pltpu.VMEM((SEG, CHUNK), jnp.int32),
            pltpu.VMEM((SEG, CHUNK), jnp.int32),
            [pltpu.VMEM((CHUNK, D3), jnp.float32) for _ in range(2)],
            pltpu.VMEM_SHARED((NPAD, D3), jnp.float32),
            pltpu.VMEM_SHARED((NPAD, D3), jnp.float32),
            [pltpu.SemaphoreType.DMA for _ in range(2)],
        ],
    )
    def agg(z_hbm, src_hbm, dst_hbm, out_hbm,
            src_v, dst_v, rows_v, hs, acc, gsem):
        c = lax.axis_index("c")
        s = lax.axis_index("s")
        base = (c * NSUB + s) * CPW_FAST
        r0 = s * RPS
        ngrp = SEG // 2

        def gather(j, b):
            pltpu.async_copy(hs.at[src_v.at[j]], rows_v[b], gsem[b])

        def gather_wait(b):
            pltpu.make_async_copy(
                hs.at[src_v.at[0]], rows_v[b], gsem[b]).wait()

        for p in range(npass):
            def zrow(r, carry):
                for k in range(D3 // 16):
                    rows_v[0][r, pl.ds(k * 16, 16)] = jnp.zeros((16,),
                                                                jnp.float32)
                return carry

            lax.fori_loop(0, CHUNK, zrow, 0)
            if npass == 1:
                pltpu.sync_copy(z_hbm.at[pl.ds(r0, RPS)],
                                hs.at[pl.ds(r0, RPS)])
            else:
                pltpu.sync_copy(
                    z_hbm.at[pl.ds(r0, RPS), pl.ds(p * D3, D3)],
                    hs.at[pl.ds(r0, RPS)])
            full = RPS // CHUNK
            for q in range(full):
                pltpu.sync_copy(rows_v[0],
                                acc.at[pl.ds(r0 + q * CHUNK, CHUNK)])
            rem = RPS - full * CHUNK
            if rem:
                pltpu.sync_copy(rows_v[0].at[pl.ds(0, rem)],
                                acc.at[pl.ds(r0 + full * CHUNK, rem)])
            plsc.subcore_barrier()

            for seg_i in range(CPW_FAST // SEG):
                pltpu.sync_copy(
                    src_hbm.at[pl.ds(base + seg_i * SEG, SEG)], src_v)
                pltpu.sync_copy(
                    dst_hbm.at[pl.ds(base + seg_i * SEG, SEG)], dst_v)
                for b in range(2):
                    gather(b, b)

                def body(g, carry):
                    for b in range(2):
                        gather_wait(b)
                        pltpu.sync_copy(rows_v[b],
                                        acc.at[dst_v.at[g * 2 + b]],
                                        add=True)

                        @pl.when(g < ngrp - 1)
                        def _():
                            gather((g + 1) * 2 + b, b)
                    return carry

                lax.fori_loop(0, ngrp, body, 0)
            plsc.subcore_barrier()
            if npass == 1:
                pltpu.sync_copy(acc.at[pl.ds(r0, RPS)],
                                out_hbm.at[pl.ds(c * NPAD + r0, RPS)])
            else:
                pltpu.sync_copy(
                    acc.at[pl.ds(r0, RPS)],
                    out_hbm.at[pl.ds(c * NPAD + r0, RPS), pl.ds(p * D3, D3)])

    return agg


_agg128 = _make_agg(D_H)
_agg64 = _make_agg(D3)


@functools.partial(
    pl.kernel,
    out_type=(jax.ShapeDtypeStruct((2 * NPAD, 16), jnp.float32),
              jax.ShapeDtypeStruct((2 * NPAD, 16), jnp.float32)),
    mesh=_MESH,
    compiler_params=pltpu.CompilerParams(use_tc_tiling_on_sc=False),
    scratch_types=[
        pltpu.VMEM((SEG, CHUNK), jnp.int32),
        pltpu.VMEM((SEG, CHUNK), jnp.int32),
        pltpu.VMEM((CHUNK, 16), jnp.float32),
        pltpu.VMEM_SHARED((NPAD, 16), jnp.float32),
        pltpu.VMEM_SHARED((NPAD, 16), jnp.float32),
        [pltpu.SemaphoreType.DMA for _ in range(NBUF)],
        [pltpu.SemaphoreType.DMA for _ in range(NBUF)],
    ],
)
def _deg_kernel(src_hbm, dst_hbm, zeros_hbm, ones_hbm,
                outdeg_hbm, indeg_hbm,
                src_v, dst_v, ones_v, acc_a, acc_b, asem, bsem):
    c = lax.axis_index("c")
    s = lax.axis_index("s")
    base = (c * NSUB + s) * CPW_FAST
    r0 = s * RPS
    pltpu.sync_copy(zeros_hbm.at[pl.ds(r0, RPS)], acc_a.at[pl.ds(r0, RPS)])
    pltpu.sync_copy(zeros_hbm.at[pl.ds(r0, RPS)], acc_b.at[pl.ds(r0, RPS)])
    pltpu.sync_copy(ones_hbm, ones_v)
    plsc.subcore_barrier()

    def run_segment(segbase):
        pltpu.sync_copy(src_hbm.at[pl.ds(segbase, SEG)], src_v)
        pltpu.sync_copy(dst_hbm.at[pl.ds(segbase, SEG)], dst_v)

        def body(g, carry):
            for b in range(NBUF):
                @pl.when(g > 0)
                def _():
                    pltpu.make_async_copy(
                        ones_v, acc_a.at[src_v.at[0]], asem[b]).wait()
                    pltpu.make_async_copy(
                        ones_v, acc_b.at[dst_v.at[0]], bsem[b]).wait()
                j = g * NBUF + b
                pltpu.async_copy(ones_v, acc_a.at[src_v.at[j]], asem[b],
                                 add=True)
                pltpu.async_copy(ones_v, acc_b.at[dst_v.at[j]], bsem[b],
                                 add=True)
            return carry

        lax.fori_loop(0, SEG // NBUF, body, 0)
        for b in range(NBUF):
            pltpu.make_async_copy(
                ones_v, acc_a.at[src_v.at[0]], asem[b]).wait()
            pltpu.make_async_copy(
                ones_v, acc_b.at[dst_v.at[0]], bsem[b]).wait()

    for seg_i in range(FSEG):
        run_segment(base + seg_i * SEG)
    plsc.subcore_barrier()
    pltpu.sync_copy(acc_a.at[pl.ds(r0, RPS)],
                    outdeg_hbm.at[pl.ds(c * NPAD + r0, RPS)])
    pltpu.sync_copy(acc_b.at[pl.ds(r0, RPS)],
                    indeg_hbm.at[pl.ds(c * NPAD + r0, RPS)])


_PREC = jax.lax.Precision.HIGHEST


def _tc_norms_body(degs_ref, degd_ref, ns_ref, nd_ref):
    out_deg = degs_ref[:NPAD, 0:1] + degs_ref[NPAD:, 0:1]
    in_deg = degd_ref[:NPAD, 0:1] + degd_ref[NPAD:, 0:1]
    ns_ref[...] = lax.rsqrt(jnp.maximum(out_deg, 1.0))
    nd_ref[...] = lax.rsqrt(jnp.maximum(in_deg, 1.0))


def _tc_norms(degs, degd):
    return pl.pallas_call(
        _tc_norms_body,
        out_shape=(jax.ShapeDtypeStruct((NPAD, 1), jnp.float32),
                   jax.ShapeDtypeStruct((NPAD, 1), jnp.float32)),
    )(degs, degd)


def _tc_z1_body(x_ref, w1_ref, ns_ref, z_ref):
    z = jnp.dot(x_ref[...], w1_ref[...],
                preferred_element_type=jnp.float32, precision=_PREC)
    z_ref[...] = z * ns_ref[...]


def _tc_z1(x, w1, ns):
    return pl.pallas_call(
        _tc_z1_body,
        out_shape=jax.ShapeDtypeStruct((NPAD, D_H), jnp.float32),
    )(x, w1, ns)


def _tc_mid_body(agg_ref, nd_ref, b_ref, w_ref, ns_ref, z_ref):
    agg = agg_ref[:NPAD, :] + agg_ref[NPAD:, :]
    h = jnp.maximum(agg * nd_ref[...] + b_ref[...][None, :], 0.0)
    mask = lax.broadcasted_iota(jnp.int32, (NPAD, 1), 0) < N
    h = jnp.where(mask, h, 0.0)
    mu = jnp.sum(h) / (N * D_H)
    d = h - mu
    var = jnp.sum(jnp.where(mask, d * d, 0.0)) / (N * D_H)
    hn = jnp.where(mask, d * lax.rsqrt(var + 1e-5), 0.0)
    z = jnp.dot(hn, w_ref[...],
                preferred_element_type=jnp.float32, precision=_PREC)
    z_ref[...] = z * ns_ref[...]


def _tc_mid(agg, nd, b, w, ns):
    return pl.pallas_call(
        _tc_mid_body,
        out_shape=jax.ShapeDtypeStruct((NPAD, w.shape[1]), jnp.float32),
    )(agg, nd, b, w, ns)


def _tc_final_body(agg_ref, nd_ref, b_ref, out_ref):
    agg = agg_ref[:NPAD, :] + agg_ref[NPAD:, :]
    out_ref[...] = agg * nd_ref[...] + b_ref[...][None, :]


def _tc_final(agg, nd, b):
    return pl.pallas_call(
        _tc_final_body,
        out_shape=jax.ShapeDtypeStruct((NPAD, D3), jnp.float32),
    )(agg, nd, b)


def kernel(features, edge_index, W1, b1, W2, b2, W3, b3):
    src = edge_index[0]
    dst = edge_index[1]
    pad = jnp.full((EPAD - E,), N, dtype=jnp.int32)
    src3 = jnp.concatenate([src, pad]).reshape(TOTALC, CHUNK)
    dst3 = jnp.concatenate([dst, pad]).reshape(TOTALC, CHUNK)

    x = jnp.zeros((NPAD, D_IN), jnp.float32).at[:N].set(features)
    zeros16 = jnp.zeros((NPAD, 16), jnp.float32)
    ones16 = jnp.ones((CHUNK, 16), jnp.float32)
    W3p = jnp.zeros((D_H, D3), jnp.float32).at[:, :D_OUT].set(W3)
    b3p = jnp.zeros((D3,), jnp.float32).at[:D_OUT].set(b3)

    degs, degd = _deg_kernel(src3, dst3, zeros16, ones16)
    ns, nd = _tc_norms(degs, degd)
    z1 = _tc_z1(x, W1, ns)
    a1 = _agg128(z1, src3, dst3)
    z2 = _tc_mid(a1, nd, b1, W2, ns)
    a2 = _agg128(z2, src3, dst3)
    z3 = _tc_mid(a2, nd, b2, W3p, ns)
    a3 = _agg64(z3, src3, dst3)
    outp = _tc_final(a3, nd, b3p)
    return outp[:N, :D_OUT]


# speedup vs baseline: 2.5848x; 1.0002x over previous
import functools

import jax
import jax.numpy as jnp
from jax import lax
from jax.experimental import pallas as pl
from jax.experimental.pallas import tpu as pltpu
from jax.experimental.pallas import tpu_sc as plsc

N = 10000
E = 320000
D_IN = 128
D_H = 128
D_OUT = 40
D3 = 64

NCORE = 2
NSUB = 16
CHUNK = 256
NBUF = 4
SEG = 20
FSEG = 2
CPW_FAST = FSEG * SEG
TOTALC = NCORE * NSUB * CPW_FAST
EPAD = TOTALC * CHUNK
NPAD = 10112
RPS = NPAD // NSUB

_MESH = plsc.VectorSubcoreMesh(
    core_axis_name="c", subcore_axis_name="s",
    num_cores=NCORE, num_subcores=NSUB)


def _make_agg(D):
    npass = D // D3

    @functools.partial(
        pl.kernel,
        out_type=jax.ShapeDtypeStruct((2 * NPAD, D), jnp.float32),
        mesh=_MESH,
        compiler_params=pltpu.CompilerParams(use_tc_tiling_on_sc=False),
        scratch_types=[
            pltpu.VMEM((SEG, CHUNK), jnp.int32),
            pltpu.VMEM((SEG, CHUNK), jnp.int32),
            [pltpu.VMEM((CHUNK, D3), jnp.float32) for _ in range(2)],
            pltpu.VMEM_SHARED((NPAD, D3), jnp.float32),
            pltpu.VMEM_SHARED((NPAD, D3), jnp.float32),
            [pltpu.SemaphoreType.DMA for _ in range(2)],
        ],
    )
    def agg(z_hbm, src_hbm, dst_hbm, out_hbm,
            src_v, dst_v, rows_v, hs, acc, gsem):
        c = lax.axis_index("c")
        s = lax.axis_index("s")
        base = (c * NSUB + s) * CPW_FAST
        r0 = s * RPS
        ngrp = SEG // 2

        def gather(j, b):
            pltpu.async_copy(hs.at[src_v.at[j]], rows_v[b], gsem[b])

        def gather_wait(b):
            pltpu.make_async_copy(
                hs.at[src_v.at[0]], rows_v[b], gsem[b]).wait()

        for p in range(npass):
            def zrow(r, carry):
                for k in range(D3 // 16):
                    rows_v[0][r, pl.ds(k * 16, 16)] = jnp.zeros((16,),
                                                                jnp.float32)
                return carry

            lax.fori_loop(0, CHUNK, zrow, 0)
            if npass == 1:
                pltpu.sync_copy(z_hbm.at[pl.ds(r0, RPS)],
                                hs.at[pl.ds(r0, RPS)])
            else:
                pltpu.sync_copy(
                    z_hbm.at[pl.ds(r0, RPS), pl.ds(p * D3, D3)],
                    hs.at[pl.ds(r0, RPS)])
            full = RPS // CHUNK
            for q in range(full):
                pltpu.sync_copy(rows_v[0],
                                acc.at[pl.ds(r0 + q * CHUNK, CHUNK)])
            rem = RPS - full * CHUNK
            if rem:
                pltpu.sync_copy(rows_v[0].at[pl.ds(0, rem)],
                                acc.at[pl.ds(r0 + full * CHUNK, rem)])
            plsc.subcore_barrier()

            for seg_i in range(CPW_FAST // SEG):
                pltpu.sync_copy(
                    src_hbm.at[pl.ds(base + seg_i * SEG, SEG)], src_v)
                pltpu.sync_copy(
                    dst_hbm.at[pl.ds(base + seg_i * SEG, SEG)], dst_v)
                for b in range(2):
                    gather(b, b)

                def body(g, carry):
                    for b in range(2):
                        gather_wait(b)
                        pltpu.sync_copy(rows_v[b],
                                        acc.at[dst_v.at[g * 2 + b]],
                                        add=True)

                        @pl.when(g < ngrp - 1)
                        def _():
                            gather((g + 1) * 2 + b, b)
                    return carry

                lax.fori_loop(0, ngrp, body, 0)
            plsc.subcore_barrier()
            if npass == 1:
                pltpu.sync_copy(acc.at[pl.ds(r0, RPS)],
                                out_hbm.at[pl.ds(c * NPAD + r0, RPS)])
            else:
                pltpu.sync_copy(
                    acc.at[pl.ds(r0, RPS)],
                    out_hbm.at[pl.ds(c * NPAD + r0, RPS), pl.ds(p * D3, D3)])

    return agg


_agg128 = _make_agg(D_H)
_agg64 = _make_agg(D3)


@functools.partial(
    pl.kernel,
    out_type=(jax.ShapeDtypeStruct((2 * NPAD, 16), jnp.float32),
              jax.ShapeDtypeStruct((2 * NPAD, 16), jnp.float32)),
    mesh=_MESH,
    compiler_params=pltpu.CompilerParams(use_tc_tiling_on_sc=False),
    scratch_types=[
        pltpu.VMEM((SEG, CHUNK), jnp.int32),
        pltpu.VMEM((SEG, CHUNK), jnp.int32),
        pltpu.VMEM((CHUNK, 16), jnp.float32),
        pltpu.VMEM_SHARED((NPAD, 16), jnp.float32),
        pltpu.VMEM_SHARED((NPAD, 16), jnp.float32),
        [pltpu.SemaphoreType.DMA for _ in range(NBUF)],
        [pltpu.SemaphoreType.DMA for _ in range(NBUF)],
    ],
)
def _deg_kernel(src_hbm, dst_hbm, zeros_hbm, ones_hbm,
                outdeg_hbm, indeg_hbm,
                src_v, dst_v, ones_v, acc_a, acc_b, asem, bsem):
    c = lax.axis_index("c")
    s = lax.axis_index("s")
    base = (c * NSUB + s) * CPW_FAST
    r0 = s * RPS
    pltpu.sync_copy(zeros_hbm.at[pl.ds(r0, RPS)], acc_a.at[pl.ds(r0, RPS)])
    pltpu.sync_copy(zeros_hbm.at[pl.ds(r0, RPS)], acc_b.at[pl.ds(r0, RPS)])
    pltpu.sync_copy(ones_hbm, ones_v)
    plsc.subcore_barrier()

    def run_segment(segbase):
        pltpu.sync_copy(src_hbm.at[pl.ds(segbase, SEG)], src_v)
        pltpu.sync_copy(dst_hbm.at[pl.ds(segbase, SEG)], dst_v)

        def body(g, carry):
            for b in range(NBUF):
                @pl.when(g > 0)
                def _():
                    pltpu.make_async_copy(
                        ones_v, acc_a.at[src_v.at[0]], asem[b]).wait()
                    pltpu.make_async_copy(
                        ones_v, acc_b.at[dst_v.at[0]], bsem[b]).wait()
                j = g * NBUF + b
                pltpu.async_copy(ones_v, acc_a.at[src_v.at[j]], asem[b],
                                 add=True)
                pltpu.async_copy(ones_v, acc_b.at[dst_v.at[j]], bsem[b],
                                 add=True)
            return carry

        lax.fori_loop(0, SEG // NBUF, body, 0)
        for b in range(NBUF):
            pltpu.make_async_copy(
                ones_v, acc_a.at[src_v.at[0]], asem[b]).wait()
            pltpu.make_async_copy(
                ones_v, acc_b.at[dst_v.at[0]], bsem[b]).wait()

    for seg_i in range(FSEG):
        run_segment(base + seg_i * SEG)
    plsc.subcore_barrier()
    pltpu.sync_copy(acc_a.at[pl.ds(r0, RPS)],
                    outdeg_hbm.at[pl.ds(c * NPAD + r0, RPS)])
    pltpu.sync_copy(acc_b.at[pl.ds(r0, RPS)],
                    indeg_hbm.at[pl.ds(c * NPAD + r0, RPS)])


_PREC = jax.lax.Precision.HIGHEST


def _tc_norms_body(degs_ref, degd_ref, ns_ref, nd_ref):
    out_deg = degs_ref[:NPAD, 0:1] + degs_ref[NPAD:, 0:1]
    in_deg = degd_ref[:NPAD, 0:1] + degd_ref[NPAD:, 0:1]
    ns_ref[...] = lax.rsqrt(jnp.maximum(out_deg, 1.0))
    nd_ref[...] = lax.rsqrt(jnp.maximum(in_deg, 1.0))


def _tc_norms(degs, degd):
    return pl.pallas_call(
        _tc_norms_body,
        out_shape=(jax.ShapeDtypeStruct((NPAD, 1), jnp.float32),
                   jax.ShapeDtypeStruct((NPAD, 1), jnp.float32)),
    )(degs, degd)


def _tc_z1_body(x_ref, w1_ref, ns_ref, z_ref):
    z = jnp.dot(x_ref[...], w1_ref[...],
                preferred_element_type=jnp.float32, precision=_PREC)
    z_ref[...] = z * ns_ref[...]


def _tc_z1(x, w1, ns):
    return pl.pallas_call(
        _tc_z1_body,
        out_shape=jax.ShapeDtypeStruct((NPAD, D_H), jnp.float32),
    )(x, w1, ns)


def _tc_mid_body(agg_ref, nd_ref, b_ref, w_ref, ns_ref, z_ref):
    agg = agg_ref[:NPAD, :] + agg_ref[NPAD:, :]
    h = jnp.maximum(agg * nd_ref[...] + b_ref[...][None, :], 0.0)
    mask = lax.broadcasted_iota(jnp.int32, (NPAD, 1), 0) < N
    h = jnp.where(mask, h, 0.0)
    mu = jnp.sum(h) / (N * D_H)
    d = h - mu
    var = jnp.sum(jnp.where(mask, d * d, 0.0)) / (N * D_H)
    hn = jnp.where(mask, d * lax.rsqrt(var + 1e-5), 0.0)
    z = jnp.dot(hn, w_ref[...],
                preferred_element_type=jnp.float32, precision=_PREC)
    z_ref[...] = z * ns_ref[...]


def _tc_mid(agg, nd, b, w, ns):
    return pl.pallas_call(
        _tc_mid_body,
        out_shape=jax.ShapeDtypeStruct((NPAD, w.shape[1]), jnp.float32),
    )(agg, nd, b, w, ns)


def _tc_final_body(agg_ref, nd_ref, b_ref, out_ref):
    agg = agg_ref[:NPAD, :] + agg_ref[NPAD:, :]
    out_ref[...] = agg * nd_ref[...] + b_ref[...][None, :]


def _tc_final(agg, nd, b):
    return pl.pallas_call(
        _tc_final_body,
        out_shape=jax.ShapeDtypeStruct((NPAD, D3), jnp.float32),
    )(agg, nd, b)


def kernel(features, edge_index, W1, b1, W2, b2, W3, b3):
    src = edge_index[0]
    dst = edge_index[1]
    pad = jnp.full((EPAD - E,), N, dtype=jnp.int32)
    src3 = jnp.concatenate([src, pad]).reshape(TOTALC, CHUNK)
    dst3 = jnp.concatenate([dst, pad]).reshape(TOTALC, CHUNK)

    x = jnp.zeros((NPAD, D_IN), jnp.float32).at[:N].set(features)
    zeros16 = jnp.zeros((NPAD, 16), jnp.float32)
    ones16 = jnp.ones((CHUNK, 16), jnp.float32)
    W3p = jnp.zeros((D_H, D3), jnp.float32).at[:, :D_OUT].set(W3)
    b3p = jnp.zeros((D3,), jnp.float32).at[:D_OUT].set(b3)

    degs, degd = _deg_kernel(src3, dst3, zeros16, ones16)
    ns, nd = _tc_norms(degs, degd)
    z1 = _tc_z1(x, W1, ns)
    a1 = _agg128(z1, src3, dst3)
    z2 = _tc_mid(a1, nd, b1, W2, ns)
    a2 = _agg128(z2, src3, dst3)
    z3 = _tc_mid(a2, nd, b2, W3p, ns)
    a3 = _agg64(z3, src3, dst3)
    outp = _tc_final(a3, nd, b3p)
    return outp[:N, :D_OUT]
